# baseline (speedup 1.0000x reference)
import jax
import jax.numpy as jnp
from jax.experimental import pallas as pl
from jax.experimental.pallas import tpu as pltpu

B = 8
S = 1024
D = 768
P = 4096

P_BLK = 2048
S_CHUNK = 256
N_PT = P // P_BLK
N_CHUNK = S // S_CHUNK
R_CHUNK = S_CHUNK // 8

_PREC = jax.lax.Precision.DEFAULT


def _proto_kernel(x_ref, p_ref, dist_ref, idx_ref, pn_ref):
    b = pl.program_id(0)
    pt = pl.program_id(1)

    @pl.when(jnp.logical_and(b == 0, pt == 0))
    def _():
        pall = p_ref[...]
        ss = jnp.sum(pall * pall, axis=1, keepdims=True)
        pn_ref[...] = pall * jax.lax.rsqrt(jnp.maximum(ss, 1e-24))

    pn = pn_ref[pl.ds(pt * P_BLK, P_BLK), :]

    def _mm(c):
        xc = x_ref[0, c * S_CHUNK:(c + 1) * S_CHUNK, :]
        ssx = jnp.sum(xc * xc, axis=1, keepdims=True)
        xn = xc * jax.lax.rsqrt(jnp.maximum(ssx, 1e-24))
        dots = jax.lax.dot_general(
            xn, pn,
            dimension_numbers=(((1,), (1,)), ((), ())),
            precision=_PREC,
            preferred_element_type=jnp.float32,
        )
        return dots.reshape(R_CHUNK, 8, P_BLK)

    m8 = None
    mi8 = None

    def _reduce(dr, c, m8, mi8):
        for i in range(R_CHUNK):
            di = dr[i]
            gi = c * R_CHUNK + i
            if m8 is None:
                m8 = di
                mi8 = jnp.zeros((8, P_BLK), jnp.int32)
            else:
                mask = di > m8
                m8 = jnp.where(mask, di, m8)
                mi8 = jnp.where(mask, gi, mi8)
        return m8, mi8

    prev = _mm(0)
    for c in range(1, N_CHUNK):
        cur = _mm(c)
        m8, mi8 = _reduce(prev, c - 1, m8, mi8)
        prev = cur
    m8, mi8 = _reduce(prev, N_CHUNK - 1, m8, mi8)

    row8 = mi8 * 8 + jax.lax.broadcasted_iota(jnp.int32, (8, P_BLK), 0)
    for sh in (4, 2, 1):
        m2 = pltpu.roll(m8, sh, axis=0)
        r2 = pltpu.roll(row8, sh, axis=0)
        better = (m2 > m8) | ((m2 == m8) & (r2 < row8))
        m8 = jnp.where(better, m2, m8)
        row8 = jnp.where(better, r2, row8)

    dist_ref[0] = 1.0 - m8[0:1]
    idx_ref[0] = row8[0:1]


@jax.jit
def kernel(x, prototypes):
    grid = (B, N_PT)
    dist, idx = pl.pallas_call(
        _proto_kernel,
        grid=grid,
        in_specs=[
            pl.BlockSpec((1, S, D), lambda b, pt: (b, 0, 0)),
            pl.BlockSpec((P, D), lambda b, pt: (0, 0)),
        ],
        out_specs=[
            pl.BlockSpec((1, 1, P_BLK), lambda b, pt: (b, 0, pt)),
            pl.BlockSpec((1, 1, P_BLK), lambda b, pt: (b, 0, pt)),
        ],
        out_shape=[
            jax.ShapeDtypeStruct((B, 1, P), jnp.float32),
            jax.ShapeDtypeStruct((B, 1, P), jnp.int32),
        ],
        scratch_shapes=[pltpu.VMEM((P, D), jnp.float32)],
    )(x, prototypes)
    return dist, idx.astype(jnp.int64)


# speedup vs baseline: 1.0504x; 1.0504x over previous
import jax
import jax.numpy as jnp
from jax.experimental import pallas as pl
from jax.experimental.pallas import tpu as pltpu

B = 8
S = 1024
D = 768
P = 4096

P_BLK = 4096
S_CHUNK = 256
N_PT = P // P_BLK
N_CHUNK = S // S_CHUNK
R_CHUNK = S_CHUNK // 8
P_SUB = P_BLK

_PREC = jax.lax.Precision.DEFAULT


def _proto_kernel(x_ref, p_ref, dist_ref, idx_ref, pn_ref):
    b = pl.program_id(1)

    @pl.when(b == 0)
    def _():
        pblk = p_ref[...]
        ss = jnp.sum(pblk * pblk, axis=1, keepdims=True)
        pn_ref[...] = pblk * jax.lax.rsqrt(jnp.maximum(ss, 1e-24))

    pn = pn_ref[...]

    m8 = [None] * 4
    mi8 = [None] * 4
    for c in range(N_CHUNK):
        xc = x_ref[0, c * S_CHUNK:(c + 1) * S_CHUNK, :]
        ssx = jnp.sum(xc * xc, axis=1, keepdims=True)
        xn = xc * jax.lax.rsqrt(jnp.maximum(ssx, 1e-24))
        dots = jax.lax.dot_general(
            xn, pn,
            dimension_numbers=(((1,), (1,)), ((), ())),
            precision=_PREC,
            preferred_element_type=jnp.float32,
        )
        dr = dots.reshape(R_CHUNK, 8, P_BLK)
        H = P_BLK // 4
        for h in range(4):
            for i in range(R_CHUNK):
                di = dr[i][:, h * H:(h + 1) * H]
                gi = c * R_CHUNK + i
                if m8[h] is None:
                    m8[h] = di
                    mi8[h] = jnp.zeros((8, H), jnp.int32)
                else:
                    m8n = jnp.maximum(di, m8[h])
                    mask = m8n > m8[h]
                    mi8[h] = jnp.where(mask, gi, mi8[h])
                    m8[h] = m8n

    H = P_BLK // 4
    iota8 = jax.lax.broadcasted_iota(jnp.int32, (8, H), 0)
    for h in range(4):
        mh = m8[h]
        rh = mi8[h] * 8 + iota8
        for sh in (4, 2, 1):
            m2 = pltpu.roll(mh, sh, axis=0)
            r2 = pltpu.roll(rh, sh, axis=0)
            better = (m2 > mh) | ((m2 == mh) & (r2 < rh))
            mh = jnp.where(better, m2, mh)
            rh = jnp.where(better, r2, rh)
        dist_ref[0, 0, h * H:(h + 1) * H] = 1.0 - mh[0]
        idx_ref[0, 0, h * H:(h + 1) * H] = rh[0]


@jax.jit
def kernel(x, prototypes):
    grid = (N_PT, B)
    dist, idx = pl.pallas_call(
        _proto_kernel,
        grid=grid,
        in_specs=[
            pl.BlockSpec((1, S, D), lambda pt, b: (b, 0, 0)),
            pl.BlockSpec((P_BLK, D), lambda pt, b: (pt, 0)),
        ],
        out_specs=[
            pl.BlockSpec((1, 1, P_BLK), lambda pt, b: (b, 0, pt)),
            pl.BlockSpec((1, 1, P_BLK), lambda pt, b: (b, 0, pt)),
        ],
        out_shape=[
            jax.ShapeDtypeStruct((B, 1, P), jnp.float32),
            jax.ShapeDtypeStruct((B, 1, P), jnp.int32),
        ],
        scratch_shapes=[pltpu.VMEM((P_BLK, D), jnp.float32)],
    )(x, prototypes)
    return dist, idx.astype(jnp.int64)
